# baseline (speedup 1.0000x reference)
import dataclasses
import functools

import jax
import jax.numpy as jnp
from jax import lax
from jax.experimental import pallas as pl
from jax.experimental.pallas import tpu as pltpu
from jax.experimental.pallas import tpu_sc as plsc

R = 128
C = 32768
L = 16
NSUB = 16
NW = 2 * NSUB
ROWS_PER_W = R // NW
BC = 2048
NB = C // BC
SEG = 128

_mesh = plsc.VectorSubcoreMesh(core_axis_name="c", subcore_axis_name="s")

_cp = pltpu.CompilerParams()
if "needs_layout_passes" in getattr(pltpu.CompilerParams, "__dataclass_fields__", {}):
    _cp = dataclasses.replace(_cp, needs_layout_passes=False)


@functools.partial(
    pl.kernel,
    out_type=jax.ShapeDtypeStruct((R, C), jnp.float32),
    mesh=_mesh,
    scratch_types=[
        pltpu.VMEM((C,), jnp.float32),
        pltpu.SemaphoreType.DMA,
    ],
    compiler_params=_cp,
)
def _zeros_sc(o_hbm, zb, sem):
    wid = lax.axis_index("c") * NSUB + lax.axis_index("s")
    row0 = wid * ROWS_PER_W

    @pl.loop(0, C, step=8 * L)
    def _(i):
        for k in range(8):
            zb[pl.ds(i + k * L, L)] = jnp.zeros((L,), jnp.float32)

    cps = [pltpu.async_copy(zb, o_hbm.at[row0 + r], sem)
           for r in range(ROWS_PER_W)]
    for cp in cps:
        cp.wait()


def _argmax_body(x_ref, o_ref, vbest, ibest):
    j = pl.program_id(0)
    xb = x_ref[...]
    m = jnp.max(xb, axis=1, keepdims=True)
    ii = lax.broadcasted_iota(jnp.int32, (R, BC), 1)
    loc = jnp.min(jnp.where(xb == m, ii, BC), axis=1, keepdims=True) + j * BC

    @pl.when(j == 0)
    def _():
        vbest[...] = m
        ibest[...] = loc

    @pl.when(j > 0)
    def _():
        better = m > vbest[...]
        ibest[...] = jnp.where(better, loc, ibest[...])
        vbest[...] = jnp.where(better, m, vbest[...])

    o_ref[...] = ibest[...]


_argmax_tc = pl.pallas_call(
    _argmax_body,
    grid=(NB,),
    in_specs=[pl.BlockSpec((R, BC), lambda j: (0, j))],
    out_specs=pl.BlockSpec((R, 1), lambda j: (0, 0)),
    out_shape=jax.ShapeDtypeStruct((R, 1), jnp.int32),
    scratch_shapes=[
        pltpu.VMEM((R, 1), jnp.float32),
        pltpu.VMEM((R, 1), jnp.int32),
    ],
)


def _patch_body(idx_smem, idx_vmem, z_ref, o_ref, seg_vmem, sem):
    lane = lax.broadcasted_iota(jnp.int32, (R, SEG), 1)
    pos = idx_vmem[...] % SEG
    seg_vmem[...] = jnp.where(lane == pos, 1.0, 0.0).astype(jnp.float32)

    def issue(r, _):
        col = (idx_smem[r, 0] // SEG) * SEG
        pltpu.make_async_copy(
            seg_vmem.at[pl.ds(r, 1)],
            o_ref.at[pl.ds(r, 1), pl.ds(col, SEG)],
            sem,
        ).start()
        return 0

    lax.fori_loop(0, R, issue, 0)

    def drain(r, _):
        col = (idx_smem[r, 0] // SEG) * SEG
        pltpu.make_async_copy(
            seg_vmem.at[pl.ds(r, 1)],
            o_ref.at[pl.ds(r, 1), pl.ds(col, SEG)],
            sem,
        ).wait()
        return 0

    lax.fori_loop(0, R, drain, 0)


_patch_tc = pl.pallas_call(
    _patch_body,
    in_specs=[
        pl.BlockSpec(memory_space=pltpu.SMEM),
        pl.BlockSpec(memory_space=pltpu.VMEM),
        pl.BlockSpec(memory_space=pl.ANY),
    ],
    out_specs=pl.BlockSpec(memory_space=pl.ANY),
    out_shape=jax.ShapeDtypeStruct((R, C), jnp.float32),
    scratch_shapes=[
        pltpu.VMEM((R, SEG), jnp.float32),
        pltpu.SemaphoreType.DMA,
    ],
    input_output_aliases={2: 0},
)


def kernel(x):
    z = _zeros_sc()
    idx = _argmax_tc(x)
    return _patch_tc(idx, idx, z)


# speedup vs baseline: 1.1552x; 1.1552x over previous
import dataclasses
import functools

import jax
import jax.numpy as jnp
from jax import lax
from jax.experimental import pallas as pl
from jax.experimental.pallas import tpu as pltpu
from jax.experimental.pallas import tpu_sc as plsc

R = 128
C = 32768
L = 16
NSUB = 16
NW = 2 * NSUB
ROWS_PER_W = R // NW
GROUP = 16
GSIZE = GROUP * L
NGROUPS = C // GSIZE
IMAX = 2147483647

_mesh = plsc.VectorSubcoreMesh(core_axis_name="c", subcore_axis_name="s")

_cp = pltpu.CompilerParams()
if "needs_layout_passes" in getattr(pltpu.CompilerParams, "__dataclass_fields__", {}):
    _cp = dataclasses.replace(_cp, needs_layout_passes=False)


def _row_argmax(buf):
    iota = lax.iota(jnp.int32, L)

    def gbody(g, carry):
        best, bgrp = carry
        base = g * GSIZE
        vs = [buf[pl.ds(base + k * L, L)] for k in range(GROUP)]
        while len(vs) > 1:
            vs = [jnp.maximum(vs[i], vs[i + 1]) for i in range(0, len(vs), 2)]
        gm = vs[0]
        better = gm > best
        best = jnp.where(better, gm, best)
        bgrp = jnp.where(better, g, bgrp)
        return best, bgrp

    best, bgrp = lax.fori_loop(
        0, NGROUPS, gbody,
        (jnp.full((L,), -jnp.inf, jnp.float32), jnp.zeros((L,), jnp.int32)))

    m = jnp.max(best)
    gstar = jnp.min(jnp.where(best == m, bgrp, jnp.int32(IMAX)))
    base = gstar * GSIZE
    acc = jnp.full((L,), IMAX, jnp.int32)
    for k in range(GROUP):
        off = base + k * L
        v = buf[pl.ds(off, L)]
        acc = jnp.minimum(acc, jnp.where(v == m, iota + off, jnp.int32(IMAX)))
    return jnp.min(acc)


@functools.partial(
    pl.kernel,
    out_type=jax.ShapeDtypeStruct((R, C), jnp.float32),
    mesh=_mesh,
    scratch_types=[
        pltpu.VMEM((C,), jnp.float32),
        pltpu.VMEM((C,), jnp.float32),
        pltpu.VMEM((C,), jnp.float32),
        pltpu.SemaphoreType.DMA,
        pltpu.SemaphoreType.DMA,
        pltpu.SemaphoreType.DMA,
    ],
    compiler_params=_cp,
)
def _onehot_sc(x_hbm, o_hbm, buf0, buf1, ob, sem0, sem1, osem):
    wid = lax.axis_index("c") * NSUB + lax.axis_index("s")
    row0 = wid * ROWS_PER_W

    bufs = (buf0, buf1)
    sems = (sem0, sem1)
    in_cp = [None] * ROWS_PER_W
    in_cp[0] = pltpu.async_copy(x_hbm.at[row0], buf0, sem0)

    @pl.loop(0, C, step=GSIZE)
    def _(i):
        for k in range(GROUP):
            ob[pl.ds(i + k * L, L)] = jnp.zeros((L,), jnp.float32)

    iota = lax.iota(jnp.int32, L)
    prev_start = jnp.int32(0)
    out_cp = None
    for r in range(ROWS_PER_W):
        if r + 1 < ROWS_PER_W:
            in_cp[r + 1] = pltpu.async_copy(
                x_hbm.at[row0 + r + 1], bufs[(r + 1) % 2], sems[(r + 1) % 2])
        in_cp[r].wait()
        idx = _row_argmax(bufs[r % 2])
        start = (idx // L) * L
        pos = idx - start
        if out_cp is not None:
            out_cp.wait()
        ob[pl.ds(prev_start, L)] = jnp.zeros((L,), jnp.float32)
        ob[pl.ds(start, L)] = jnp.where(iota == pos, 1.0, 0.0).astype(jnp.float32)
        out_cp = pltpu.async_copy(ob, o_hbm.at[row0 + r], osem)
        prev_start = start
    out_cp.wait()


def kernel(x):
    return _onehot_sc(x)
